# baseline (speedup 1.0000x reference)
import functools

import jax
import jax.numpy as jnp
from jax import lax
from jax.experimental import pallas as pl
from jax.experimental.pallas import tpu as pltpu
from jax.experimental.pallas import tpu_sc as plsc

NC = 2
NS = 16
L = 16
NW = NC * NS

B = 16384
PF = 32
B_PER_W = B // NW
GROUPS = B_PER_W // L


def _make_kernel():
  mesh = plsc.VectorSubcoreMesh(
      core_axis_name="c", subcore_axis_name="s", num_cores=NC, num_subcores=NS
  )

  @functools.partial(
      pl.kernel,
      out_type=jax.ShapeDtypeStruct((B,), jnp.float32),
      mesh=mesh,
      scratch_types=[
          pltpu.VMEM((B_PER_W,), jnp.int32),
          pltpu.VMEM((B_PER_W,), jnp.int32),
          pltpu.VMEM((B_PER_W, PF), jnp.float32),
          pltpu.VMEM((B_PER_W, PF), jnp.float32),
          pltpu.VMEM((1, PF), jnp.float32),
          pltpu.VMEM((B_PER_W,), jnp.float32),
          pltpu.SemaphoreType.DMA,
          pltpu.SemaphoreType.DMA,
      ],
      compiler_params=pltpu.CompilerParams(
          needs_layout_passes=False, use_tc_tiling_on_sc=False
      ),
  )
  def gmf_kernel(users_hbm, movies_hbm, utab_hbm, mtab_hbm, w_hbm, out_hbm,
                 uidx_v, midx_v, urows_v, mrows_v, w_v, res_v, usem, msem):
    wid = lax.axis_index("s") * NC + lax.axis_index("c")
    base = wid * B_PER_W

    pltpu.sync_copy(users_hbm.at[pl.ds(base, B_PER_W)], uidx_v)
    pltpu.sync_copy(movies_hbm.at[pl.ds(base, B_PER_W)], midx_v)
    cu = pltpu.async_copy(utab_hbm.at[uidx_v], urows_v, usem)
    cm = pltpu.async_copy(mtab_hbm.at[midx_v], mrows_v, msem)
    pltpu.sync_copy(w_hbm, w_v)
    w0 = w_v[0, pl.ds(0, L)]
    w1 = w_v[0, pl.ds(L, L)]
    lane = lax.iota(jnp.int32, L)
    cu.wait()
    cm.wait()

    def group_body(g, _):
      acc = jnp.zeros((L,), jnp.float32)
      row0 = g * L
      for j in range(L):
        i = row0 + j
        t = (urows_v[i, pl.ds(0, L)] * mrows_v[i, pl.ds(0, L)] * w0
             + urows_v[i, pl.ds(L, L)] * mrows_v[i, pl.ds(L, L)] * w1)
        s = jnp.sum(t)
        acc = jnp.where(lane == j, s, acc)
      res_v[pl.ds(row0, L)] = acc
      return ()

    lax.fori_loop(0, GROUPS, group_body, ())
    pltpu.sync_copy(res_v, out_hbm.at[pl.ds(base, B_PER_W)])

  return gmf_kernel


_gmf = _make_kernel()


@jax.jit
def kernel(users, movies, user_table, movie_table, W):
  out = _gmf(users, movies, user_table, movie_table, W)
  return out.reshape(B, 1)
